# Initial kernel scaffold; baseline (speedup 1.0000x reference)
import jax
import jax.numpy as jnp
from jax.experimental import pallas as pl


def kernel(hidden_states, cos, sin, ln1_w, ln2_w, wqkv, out_w, router_w, w1, v1, w2):
    raise NotImplementedError("write your pallas kernel here")


# speedup vs baseline: 1.5997x; 1.5997x over previous
import functools
import jax
import jax.numpy as jnp
from jax import lax
from jax.experimental import pallas as pl
from jax.experimental.pallas import tpu as pltpu

S = 2048
D = 768
N_HEADS = 12
KV_HEADS = 4
HEAD_DIM = 64
D_FF = 1536
N_EXPERTS = 8
TOP_K = 2
EPS = 1e-5
ROW_T = 256
MOE_T = 256
N_TILES_MAX = S * TOP_K // MOE_T + N_EXPERTS
PAD_ROWS = N_TILES_MAX * MOE_T
NEG = -1e30


def _ln(x, w):
    mu = jnp.mean(x, axis=-1, keepdims=True)
    var = jnp.mean((x - mu) ** 2, axis=-1, keepdims=True)
    return (x - mu) * jax.lax.rsqrt(var + EPS) * w


def _k1_body(x_ref, w_ref, ln1_ref, cos_ref, sin_ref, q_ref, k_ref, v_ref):
    h = _ln(x_ref[...], ln1_ref[...])
    qkv = jax.lax.dot_general(h, w_ref[...], (((1,), (0,)), ((), ())),
                              preferred_element_type=jnp.float32)
    q = qkv[:, : N_HEADS * HEAD_DIM]
    k = qkv[:, N_HEADS * HEAD_DIM : (N_HEADS + KV_HEADS) * HEAD_DIM]
    v = qkv[:, (N_HEADS + KV_HEADS) * HEAD_DIM :]

    def rope(x, nh):
        cos_t = jnp.concatenate([cos_ref[...]] * nh, axis=1)
        sin_t = jnp.concatenate([sin_ref[...]] * nh, axis=1)
        lane = jax.lax.broadcasted_iota(jnp.int32, x.shape, 1) % HEAD_DIM
        zp = pltpu.roll(x, x.shape[1] - HEAD_DIM // 2, 1)
        zm = pltpu.roll(x, HEAD_DIM // 2, 1)
        rot = jnp.where(lane < HEAD_DIM // 2, -zp, zm)
        return x * cos_t + rot * sin_t

    q_ref[...] = rope(q, N_HEADS)
    k_ref[...] = rope(k, KV_HEADS)
    v_ref[...] = v


def _k1(x, wqkv, ln1_w, cos, sin):
    n = S // ROW_T
    return pl.pallas_call(
        _k1_body,
        grid=(n,),
        in_specs=[
            pl.BlockSpec((ROW_T, D), lambda i: (i, 0)),
            pl.BlockSpec((D, (N_HEADS + 2 * KV_HEADS) * HEAD_DIM), lambda i: (0, 0)),
            pl.BlockSpec((1, D), lambda i: (0, 0)),
            pl.BlockSpec((ROW_T, HEAD_DIM), lambda i: (i, 0)),
            pl.BlockSpec((ROW_T, HEAD_DIM), lambda i: (i, 0)),
        ],
        out_specs=[
            pl.BlockSpec((ROW_T, N_HEADS * HEAD_DIM), lambda i: (i, 0)),
            pl.BlockSpec((ROW_T, KV_HEADS * HEAD_DIM), lambda i: (i, 0)),
            pl.BlockSpec((ROW_T, KV_HEADS * HEAD_DIM), lambda i: (i, 0)),
        ],
        out_shape=[
            jax.ShapeDtypeStruct((S, N_HEADS * HEAD_DIM), jnp.float32),
            jax.ShapeDtypeStruct((S, KV_HEADS * HEAD_DIM), jnp.float32),
            jax.ShapeDtypeStruct((S, KV_HEADS * HEAD_DIM), jnp.float32),
        ],
    )(x, wqkv, ln1_w, cos, sin)


def _k2_body(q_ref, k_ref, v_ref, o_ref):
    qt = pl.program_id(0)
    rep = N_HEADS // KV_HEADS
    for h in range(N_HEADS):
        q = q_ref[:, h * HEAD_DIM : (h + 1) * HEAD_DIM]
        kv = h // rep
        k = k_ref[:, kv * HEAD_DIM : (kv + 1) * HEAD_DIM]
        v = v_ref[:, kv * HEAD_DIM : (kv + 1) * HEAD_DIM]
        s = jax.lax.dot_general(q, k, (((1,), (1,)), ((), ())),
                                preferred_element_type=jnp.float32)
        s = s * (1.0 / (HEAD_DIM ** 0.5))
        row = qt * ROW_T + jax.lax.broadcasted_iota(jnp.int32, s.shape, 0)
        col = jax.lax.broadcasted_iota(jnp.int32, s.shape, 1)
        s = jnp.where(col <= row, s, NEG)
        m = jnp.max(s, axis=1, keepdims=True)
        p = jnp.exp(s - m)
        p = p / jnp.sum(p, axis=1, keepdims=True)
        o_ref[:, h * HEAD_DIM : (h + 1) * HEAD_DIM] = jax.lax.dot_general(
            p, v, (((1,), (0,)), ((), ())), preferred_element_type=jnp.float32)


def _k2(q, k, v):
    return pl.pallas_call(
        _k2_body,
        grid=(S // ROW_T,),
        in_specs=[
            pl.BlockSpec((ROW_T, N_HEADS * HEAD_DIM), lambda i: (i, 0)),
            pl.BlockSpec((S, KV_HEADS * HEAD_DIM), lambda i: (0, 0)),
            pl.BlockSpec((S, KV_HEADS * HEAD_DIM), lambda i: (0, 0)),
        ],
        out_specs=pl.BlockSpec((ROW_T, N_HEADS * HEAD_DIM), lambda i: (i, 0)),
        out_shape=jax.ShapeDtypeStruct((S, N_HEADS * HEAD_DIM), jnp.float32),
    )(q, k, v)


def _k3_body(a_ref, res_ref, ow_ref, ln2_ref, rw_ref, r2_ref, h3_ref, rt_ref):
    r2 = jax.lax.dot_general(a_ref[...], ow_ref[...], (((1,), (0,)), ((), ())),
                             preferred_element_type=jnp.float32) + res_ref[...]
    r2_ref[...] = r2
    h3 = _ln(r2, ln2_ref[...])
    h3_ref[...] = h3
    logits = jax.lax.dot_general(h3, rw_ref[...], (((1,), (0,)), ((), ())),
                                 preferred_element_type=jnp.float32)
    lane = jax.lax.broadcasted_iota(jnp.int32, logits.shape, 1)
    logits = jnp.where(lane < N_EXPERTS, logits, NEG)
    l1 = jnp.max(logits, axis=1, keepdims=True)
    e1 = jnp.min(jnp.where(logits == l1, lane, 127), axis=1, keepdims=True)
    logits2 = jnp.where(lane == e1, NEG, logits)
    l2 = jnp.max(logits2, axis=1, keepdims=True)
    e2 = jnp.min(jnp.where(logits2 == l2, lane, 127), axis=1, keepdims=True)
    wa = jax.nn.sigmoid(l1 - l2)
    wb = 1.0 - wa
    rt = jnp.where(lane == 0, wa, 0.0)
    rt = jnp.where(lane == 1, wb, rt)
    rt = jnp.where(lane == 2, e1.astype(jnp.float32), rt)
    rt = jnp.where(lane == 3, e2.astype(jnp.float32), rt)
    rt_ref[...] = rt


def _k3(attn, residual, out_w, ln2_w, router_wp):
    n = S // ROW_T
    return pl.pallas_call(
        _k3_body,
        grid=(n,),
        in_specs=[
            pl.BlockSpec((ROW_T, D), lambda i: (i, 0)),
            pl.BlockSpec((ROW_T, D), lambda i: (i, 0)),
            pl.BlockSpec((D, D), lambda i: (0, 0)),
            pl.BlockSpec((1, D), lambda i: (0, 0)),
            pl.BlockSpec((D, 128), lambda i: (0, 0)),
        ],
        out_specs=[
            pl.BlockSpec((ROW_T, D), lambda i: (i, 0)),
            pl.BlockSpec((ROW_T, D), lambda i: (i, 0)),
            pl.BlockSpec((ROW_T, 128), lambda i: (i, 0)),
        ],
        out_shape=[
            jax.ShapeDtypeStruct((S, D), jnp.float32),
            jax.ShapeDtypeStruct((S, D), jnp.float32),
            jax.ShapeDtypeStruct((S, 128), jnp.float32),
        ],
    )(attn, residual, out_w, ln2_w, router_wp)


def _routing_meta(e1, e2):
    keys = jnp.stack([e1, e2], axis=1).reshape(-1)
    perm = jnp.argsort(keys, stable=True)
    keys_sorted = keys[perm]
    tok_sorted = perm // TOP_K
    counts = jnp.sum(keys[:, None] == jnp.arange(N_EXPERTS)[None, :], axis=0)
    start = jnp.concatenate([jnp.zeros((1,), jnp.int32),
                             jnp.cumsum(counts)[:-1].astype(jnp.int32)])
    ptiles = (counts + MOE_T - 1) // MOE_T
    pad_start = jnp.concatenate([jnp.zeros((1,), jnp.int32),
                                 (jnp.cumsum(ptiles)[:-1] * MOE_T).astype(jnp.int32)])
    rank = jnp.arange(S * TOP_K, dtype=jnp.int32) - start[keys_sorted]
    ppos = pad_start[keys_sorted] + rank
    rows_idx = jnp.zeros((PAD_ROWS,), jnp.int32).at[ppos].set(tok_sorted.astype(jnp.int32))
    pos_flat = jnp.zeros((S * TOP_K,), jnp.int32).at[perm].set(ppos)
    pos_a = pos_flat[0::TOP_K]
    pos_b = pos_flat[1::TOP_K]
    total = jnp.sum(ptiles).astype(jnp.int32)
    cum_end = jnp.cumsum(ptiles).astype(jnp.int32)
    t_idx = jnp.arange(N_TILES_MAX, dtype=jnp.int32)
    te = jnp.sum(cum_end[None, :] <= t_idx[:, None], axis=1).astype(jnp.int32)
    te_last = jnp.sum(cum_end <= total - 1).astype(jnp.int32)
    te = jnp.where(t_idx < total, te, te_last)
    valid = (t_idx < total).astype(jnp.int32)
    return rows_idx, pos_a, pos_b, te, valid


def _k4b_body(te_ref, va_ref, xg_ref, w1_ref, v1_ref, w2_ref, y_ref):
    t = pl.program_id(0)

    @pl.when(va_ref[t] == 1)
    def _():
        xg = xg_ref[...]
        g = jax.lax.dot_general(xg, w1_ref[0], (((1,), (1,)), ((), ())),
                                preferred_element_type=jnp.float32)
        u = jax.lax.dot_general(xg, v1_ref[0], (((1,), (1,)), ((), ())),
                                preferred_element_type=jnp.float32)
        hmid = (g * jax.nn.sigmoid(g)) * u
        y_ref[...] = jax.lax.dot_general(hmid, w2_ref[0], (((1,), (1,)), ((), ())),
                                         preferred_element_type=jnp.float32)

    @pl.when(va_ref[t] == 0)
    def _():
        y_ref[...] = jnp.zeros_like(y_ref)


def _k4b(xg, w1, v1, w2, te, valid):
    grid_spec = pltpu.PrefetchScalarGridSpec(
        num_scalar_prefetch=2,
        grid=(N_TILES_MAX,),
        in_specs=[
            pl.BlockSpec((MOE_T, D), lambda t, te, va: (t, 0)),
            pl.BlockSpec((1, D_FF, D), lambda t, te, va: (te[t], 0, 0)),
            pl.BlockSpec((1, D_FF, D), lambda t, te, va: (te[t], 0, 0)),
            pl.BlockSpec((1, D, D_FF), lambda t, te, va: (te[t], 0, 0)),
        ],
        out_specs=pl.BlockSpec((MOE_T, D), lambda t, te, va: (t, 0)),
    )
    return pl.pallas_call(
        _k4b_body,
        grid_spec=grid_spec,
        out_shape=jax.ShapeDtypeStruct((PAD_ROWS, D), jnp.float32),
    )(te, valid, xg, w1, v1, w2)


def _dispatch(h3, rows_idx):
    return h3[rows_idx]


def _combine(y, pos_a, pos_b, wa, wb):
    return wa[:, None] * y[pos_a] + wb[:, None] * y[pos_b]


NW = 32
GCHUNK = 96


def _sc_mesh():
    from jax.experimental.pallas import tpu_sc as plsc
    return plsc.VectorSubcoreMesh(core_axis_name="c", subcore_axis_name="s")


def _k4a_sc(h3, rows_idx):
    per_w = PAD_ROWS // NW
    nch = per_w // GCHUNK

    @functools.partial(
        pl.kernel,
        mesh=_sc_mesh(),
        out_type=jax.ShapeDtypeStruct((PAD_ROWS, D), jnp.float32),
        scratch_types=[
            pltpu.VMEM((GCHUNK,), jnp.int32),
            pltpu.VMEM((GCHUNK, D), jnp.float32),
            pltpu.SemaphoreType.DMA,
        ],
    )
    def body(x_hbm, idx_hbm, out_hbm, idx_v, rows_v, sem):
        wid = lax.axis_index("s") * 2 + lax.axis_index("c")
        for c in range(nch):
            base = wid * per_w + c * GCHUNK
            pltpu.sync_copy(idx_hbm.at[pl.ds(base, GCHUNK)], idx_v)
            pltpu.async_copy(x_hbm.at[idx_v], rows_v, sem).wait()
            pltpu.sync_copy(rows_v, out_hbm.at[pl.ds(base, GCHUNK)])

    return body(h3, rows_idx)


CCHUNK = 32


def _k4c_sc(y, pos_a, pos_b, wab):
    per_w = S // NW
    nch = per_w // CCHUNK

    @functools.partial(
        pl.kernel,
        mesh=_sc_mesh(),
        out_type=jax.ShapeDtypeStruct((S, D), jnp.float32),
        scratch_types=[
            pltpu.VMEM((CCHUNK,), jnp.int32),
            pltpu.VMEM((CCHUNK,), jnp.int32),
            pltpu.VMEM((2, CCHUNK), jnp.float32),
            pltpu.VMEM((CCHUNK, D), jnp.float32),
            pltpu.VMEM((CCHUNK, D), jnp.float32),
            pltpu.VMEM((CCHUNK, D), jnp.float32),
            pltpu.SemaphoreType.DMA,
        ],
    )
    def body(y_hbm, pa_hbm, pb_hbm, w_hbm, out_hbm,
             pa_v, pb_v, w_v, ya_v, yb_v, o_v, sem):
        wid = lax.axis_index("s") * 2 + lax.axis_index("c")
        for c in range(nch):
            base = wid * per_w + c * CCHUNK
            pltpu.sync_copy(pa_hbm.at[pl.ds(base, CCHUNK)], pa_v)
            pltpu.sync_copy(pb_hbm.at[pl.ds(base, CCHUNK)], pb_v)
            pltpu.sync_copy(w_hbm.at[:, pl.ds(base, CCHUNK)], w_v)
            pltpu.async_copy(y_hbm.at[pa_v], ya_v, sem).wait()
            pltpu.async_copy(y_hbm.at[pb_v], yb_v, sem).wait()

            def row(r, _):
                aw = w_v[0, r]
                bw = w_v[1, r]
                for j in range(D // 16):
                    sl = pl.ds(j * 16, 16)
                    o_v[r, sl] = aw * ya_v[r, sl] + bw * yb_v[r, sl]
                return 0

            lax.fori_loop(0, CCHUNK, row, 0)
            pltpu.sync_copy(o_v, out_hbm.at[pl.ds(base, CCHUNK)])

    return body(y, pos_a, pos_b, wab)


@jax.jit
def kernel(hidden_states, cos, sin, ln1_w, ln2_w, wqkv, out_w, router_w, w1, v1, w2):
    x = hidden_states.reshape(S, D)
    ln1 = ln1_w.reshape(1, D)
    ln2 = ln2_w.reshape(1, D)
    router_wp = jnp.zeros((D, 128), jnp.float32).at[:, :N_EXPERTS].set(router_w)

    q, k, v = _k1(x, wqkv, ln1, cos, sin)
    attn = _k2(q, k, v)
    residual2, h3, route = _k3(attn, x, out_w, ln2, router_wp)

    wa = route[:, 0]
    wb = route[:, 1]
    e1 = route[:, 2].astype(jnp.int32)
    e2 = route[:, 3].astype(jnp.int32)
    rows_idx, pos_a, pos_b, te, valid = _routing_meta(e1, e2)

    xg = _dispatch(h3, rows_idx)
    y = _k4b(xg, w1, v1, w2, te, valid)
    out = _combine(y, pos_a, pos_b, wa, wb)

    return (out.reshape(1, S, D), residual2.reshape(1, S, D))
